# Initial kernel scaffold; baseline (speedup 1.0000x reference)
import jax
import jax.numpy as jnp
from jax.experimental import pallas as pl


def kernel(x, edge_index, W1, b1, W2, b2):
    raise NotImplementedError("write your pallas kernel here")


# speedup vs baseline: 9.8982x; 9.8982x over previous
import functools

import jax
import jax.numpy as jnp
from jax import lax
from jax.experimental import pallas as pl
from jax.experimental.pallas import tpu as pltpu
from jax.experimental.pallas import tpu_sc as plsc

N = 10000
D = 128
E = 320000

NC = 2
NS = 16
NW = NC * NS
CHUNK = 128
KC = 80
EP = NW * KC * CHUNK
G = 16
NG = KC // G
NPAD = 10240
RPW = NPAD // NS
RB = 256
GRID = NPAD // RB

_mesh = plsc.VectorSubcoreMesh(core_axis_name="c", subcore_axis_name="s")


@functools.partial(
    pl.kernel,
    out_type=jax.ShapeDtypeStruct((NC, NPAD), jnp.float32),
    mesh=_mesh,
    scratch_types=[
        pltpu.VMEM((KC, CHUNK), jnp.int32),
        pltpu.VMEM((CHUNK,), jnp.float32),
        pltpu.VMEM((RPW,), jnp.float32),
        pltpu.VMEM_SHARED((NPAD,), jnp.float32),
    ],
)
def _deg_kernel(dstb_hbm, degp_hbm, idx_v, ones_v, buf_v, deg_sh):
    c = lax.axis_index("c")
    s = lax.axis_index("s")
    wid = c * NS + s
    for i in range(CHUNK // 16):
        ones_v[pl.ds(i * 16, 16)] = jnp.ones((16,), jnp.float32)
    for i in range(RPW // 16):
        buf_v[pl.ds(i * 16, 16)] = jnp.zeros((16,), jnp.float32)
    pltpu.sync_copy(buf_v, deg_sh.at[pl.ds(s * RPW, RPW)])
    plsc.subcore_barrier()
    pltpu.sync_copy(dstb_hbm.at[wid], idx_v)

    def body(j, carry):
        pltpu.sync_copy(ones_v, deg_sh.at[idx_v.at[j]], add=True)
        return carry

    lax.fori_loop(0, KC, body, 0)
    plsc.subcore_barrier()
    pltpu.sync_copy(deg_sh.at[pl.ds(s * RPW, RPW)], buf_v)
    pltpu.sync_copy(buf_v, degp_hbm.at[c, pl.ds(s * RPW, RPW)])


@functools.partial(
    pl.kernel,
    out_type=jax.ShapeDtypeStruct((NC, NPAD, D), jnp.float32),
    mesh=_mesh,
    scratch_types=[
        pltpu.VMEM((G, CHUNK), jnp.int32),
        pltpu.VMEM((G, CHUNK), jnp.int32),
        pltpu.VMEM((CHUNK, D), jnp.float32),
        pltpu.VMEM((CHUNK, D), jnp.float32),
        pltpu.VMEM_SHARED((NPAD, D), jnp.float32),
        pltpu.SemaphoreType.DMA,
        pltpu.SemaphoreType.DMA,
    ],
)
def _scatter_kernel(y_hbm, srcb_hbm, dstb_hbm, z_hbm,
                    si_v, di_v, r0, r1, z_sh, sem0, sem1):
    c = lax.axis_index("c")
    s = lax.axis_index("s")
    wid = c * NS + s

    def zrow(r, carry):
        for i in range(D // 16):
            r0[r, pl.ds(i * 16, 16)] = jnp.zeros((16,), jnp.float32)
        return carry

    lax.fori_loop(0, CHUNK, zrow, 0)
    for k in range(RPW // CHUNK):
        pltpu.sync_copy(r0, z_sh.at[pl.ds(s * RPW + k * CHUNK, CHUNK)])
    plsc.subcore_barrier()

    def group(g, carry):
        pltpu.sync_copy(srcb_hbm.at[wid, pl.ds(g * G, G)], si_v)
        pltpu.sync_copy(dstb_hbm.at[wid, pl.ds(g * G, G)], di_v)
        pltpu.async_copy(y_hbm.at[si_v.at[0]], r0, sem0)

        def body(i, carry2):
            j0 = i * 2
            j1 = j0 + 1
            pltpu.async_copy(y_hbm.at[si_v.at[j1]], r1, sem1)
            pltpu.make_async_copy(y_hbm.at[si_v.at[j0]], r0, sem0).wait()
            pltpu.sync_copy(r0, z_sh.at[di_v.at[j0]], add=True)

            @pl.when(j0 + 2 < G)
            def _():
                pltpu.async_copy(y_hbm.at[si_v.at[j0 + 2]], r0, sem0)

            pltpu.make_async_copy(y_hbm.at[si_v.at[j1]], r1, sem1).wait()
            pltpu.sync_copy(r1, z_sh.at[di_v.at[j1]], add=True)
            return carry2

        lax.fori_loop(0, G // 2, body, 0)
        return carry

    lax.fori_loop(0, NG, group, 0)
    plsc.subcore_barrier()
    for k in range(RPW // CHUNK):
        base = s * RPW + k * CHUNK
        pltpu.sync_copy(z_sh.at[pl.ds(base, CHUNK)], r0)
        pltpu.sync_copy(r0, z_hbm.at[c, pl.ds(base, CHUNK)])


def _tc1_body(xb, w1, degb, y1b, disb):
    deg = degb[0] + degb[1] + 1.0
    dis = lax.rsqrt(deg)
    xw = jnp.dot(xb[...], w1[...], preferred_element_type=jnp.float32)
    y1b[...] = xw * dis
    disb[...] = dis


def _tc2_body(zpb, y1b, disb, b1b, w2, y2b):
    z = zpb[0] + zpb[1] + y1b[...]
    h = jnp.maximum(disb[...] * z + b1b[...], 0.0)
    y2b[...] = jnp.dot(h, w2[...], preferred_element_type=jnp.float32) * disb[...]


def _tc3_body(zpb, y2b, disb, b2b, outb):
    outb[...] = disb[...] * (zpb[0] + zpb[1] + y2b[...]) + b2b[...]


_tc1 = pl.pallas_call(
    _tc1_body,
    grid=(GRID,),
    in_specs=[
        pl.BlockSpec((RB, D), lambda i: (i, 0)),
        pl.BlockSpec((D, D), lambda i: (0, 0)),
        pl.BlockSpec((NC, RB, 1), lambda i: (0, i, 0)),
    ],
    out_specs=[
        pl.BlockSpec((RB, D), lambda i: (i, 0)),
        pl.BlockSpec((RB, 1), lambda i: (i, 0)),
    ],
    out_shape=[
        jax.ShapeDtypeStruct((NPAD, D), jnp.float32),
        jax.ShapeDtypeStruct((NPAD, 1), jnp.float32),
    ],
)

_tc2 = pl.pallas_call(
    _tc2_body,
    grid=(GRID,),
    in_specs=[
        pl.BlockSpec((NC, RB, D), lambda i: (0, i, 0)),
        pl.BlockSpec((RB, D), lambda i: (i, 0)),
        pl.BlockSpec((RB, 1), lambda i: (i, 0)),
        pl.BlockSpec((1, D), lambda i: (0, 0)),
        pl.BlockSpec((D, D), lambda i: (0, 0)),
    ],
    out_specs=pl.BlockSpec((RB, D), lambda i: (i, 0)),
    out_shape=jax.ShapeDtypeStruct((NPAD, D), jnp.float32),
)

_tc3 = pl.pallas_call(
    _tc3_body,
    grid=(GRID,),
    in_specs=[
        pl.BlockSpec((NC, RB, D), lambda i: (0, i, 0)),
        pl.BlockSpec((RB, D), lambda i: (i, 0)),
        pl.BlockSpec((RB, 1), lambda i: (i, 0)),
        pl.BlockSpec((1, D), lambda i: (0, 0)),
    ],
    out_specs=pl.BlockSpec((RB, D), lambda i: (i, 0)),
    out_shape=jax.ShapeDtypeStruct((NPAD, D), jnp.float32),
)


def kernel(x, edge_index, W1, b1, W2, b2):
    ei = edge_index.astype(jnp.int32)
    src = jnp.concatenate(
        [ei[0], jnp.zeros((EP - E,), jnp.int32)]).reshape(NW, KC, CHUNK)
    dst = jnp.concatenate(
        [ei[1], jnp.full((EP - E,), N, jnp.int32)]).reshape(NW, KC, CHUNK)
    xp = jnp.concatenate([x, jnp.zeros((NPAD - N, D), x.dtype)], axis=0)
    b1r = b1.reshape(1, D)
    b2r = b2.reshape(1, D)

    degp = _deg_kernel(dst).reshape(NC, NPAD, 1)
    y1, dis = _tc1(xp, W1, degp)
    z1 = _scatter_kernel(y1, src, dst)
    y2 = _tc2(z1, y1, dis, b1r, W2)
    z2 = _scatter_kernel(y2, src, dst)
    out = _tc3(z2, y2, dis, b2r)
    return out[:N]
